# baseline (speedup 1.0000x reference)
import functools

import jax
import jax.numpy as jnp
from jax import lax
from jax.experimental import pallas as pl
from jax.experimental.pallas import tpu as pltpu
from jax.experimental.pallas import tpu_sc as plsc


def _make_sc_concat(B, C, D, NH, NL):
    mesh = plsc.VectorSubcoreMesh(core_axis_name="c", subcore_axis_name="s")

    @functools.partial(
        pl.kernel,
        mesh=mesh,
        out_type=jax.ShapeDtypeStruct((B, C + D, NH, NL), jnp.float32),
        scratch_types=[pltpu.SemaphoreType.DMA],
        compiler_params=pltpu.CompilerParams(use_tc_tiling_on_sc=False),
    )
    def sc_concat(x_hbm, p_hbm, o_hbm, sem):
        w = lax.axis_index("s") * 2 + lax.axis_index("c")
        pending = []
        for ci in range(8):
            for cj in range(8):
                cp = pltpu.make_async_copy(
                    p_hbm.at[w, ci, :, cj, :],
                    o_hbm.at[w, C + 8 * ci + cj, :, :],
                    sem,
                )
                cp.start()
                pending.append(cp)
                if len(pending) == 8:
                    for q in pending:
                        q.wait()
                    pending = []
        for c in range(C):
            cp = pltpu.make_async_copy(
                x_hbm.at[c, w // 8, :, w % 8, :],
                o_hbm.at[w, c, :, :],
                sem,
            )
            cp.start()
            pending.append(cp)
        for q in pending:
            q.wait()

    return sc_concat


def kernel(xyz, points):
    B, N, C = xyz.shape
    D = points.shape[-1]
    NH, NL = N // 128, 128
    x_v = xyz.reshape(4, 8, NH, NL, C).transpose(4, 0, 2, 1, 3)
    p_v = points.reshape(B, NH, NL, 8, 8).transpose(0, 3, 1, 4, 2)

    out = _make_sc_concat(B, C, D, NH, NL)(x_v, p_v)

    new_xyz = jnp.zeros((B, 1, C), dtype=xyz.dtype)
    new_points = out.transpose(0, 2, 3, 1).reshape(B, 1, N, C + D)
    return (new_xyz, new_points)


# speedup vs baseline: 28.8486x; 28.8486x over previous
import functools

import jax
import jax.numpy as jnp
from jax import lax
from jax.experimental import pallas as pl
from jax.experimental.pallas import tpu as pltpu
from jax.experimental.pallas import tpu_sc as plsc


def _make_sc_concat(B, C, D, NH, NL):
    mesh = plsc.VectorSubcoreMesh(core_axis_name="c", subcore_axis_name="s")
    HH = NH // 2

    @functools.partial(
        pl.kernel,
        mesh=mesh,
        out_type=jax.ShapeDtypeStruct((B, C + D, NH, NL), jnp.float32),
        scratch_types=[
            pltpu.VMEM((2, HH, 8, NL), jnp.float32),
            pltpu.VMEM((NH, NL), jnp.float32),
            pltpu.SemaphoreType.DMA,
            pltpu.SemaphoreType.DMA,
            pltpu.SemaphoreType.DMA,
            pltpu.SemaphoreType.DMA,
        ],
        compiler_params=pltpu.CompilerParams(use_tc_tiling_on_sc=False),
    )
    def sc_concat(x_hbm, p_hbm, o_hbm, pbuf, xbuf, ld0, ld1, st0, st1):
        w = lax.axis_index("s") * 2 + lax.axis_index("c")
        ld = [ld0, ld1]
        st = [st0, st1]
        n_chunks = 16

        def load(k):
            ci, h = k // 2, k % 2
            cp = pltpu.make_async_copy(
                p_hbm.at[w, ci, pl.ds(HH * h, HH), :, :],
                pbuf.at[k % 2],
                ld[k % 2],
            )
            cp.start()
            return cp

        loads = {0: load(0)}
        stores = {0: [], 1: []}
        for k in range(n_chunks):
            if k + 1 < n_chunks:
                for q in stores[(k + 1) % 2]:
                    q.wait()
                stores[(k + 1) % 2] = []
                loads[k + 1] = load(k + 1)
            loads[k].wait()
            ci, h = k // 2, k % 2
            for cj in range(8):
                cp = pltpu.make_async_copy(
                    pbuf.at[k % 2, :, cj, :],
                    o_hbm.at[w, C + 8 * ci + cj, pl.ds(HH * h, HH), :],
                    st[k % 2],
                )
                cp.start()
                stores[k % 2].append(cp)
        for c in range(C):
            cp = pltpu.make_async_copy(
                x_hbm.at[c, w // 8, :, w % 8, :], xbuf, ld0
            )
            cp.start()
            cp.wait()
            cp = pltpu.make_async_copy(xbuf, o_hbm.at[w, c, :, :], st0)
            cp.start()
            cp.wait()
        for k in (0, 1):
            for q in stores[k]:
                q.wait()

    return sc_concat


def kernel(xyz, points):
    B, N, C = xyz.shape
    D = points.shape[-1]
    NH, NL = N // 128, 128
    x_v = xyz.reshape(4, 8, NH, NL, C).transpose(4, 0, 2, 1, 3)
    p_v = points.reshape(B, NH, NL, 8, 8).transpose(0, 3, 1, 4, 2)

    out = _make_sc_concat(B, C, D, NH, NL)(x_v, p_v)

    new_xyz = jnp.zeros((B, 1, C), dtype=xyz.dtype)
    new_points = out.transpose(0, 2, 3, 1).reshape(B, 1, N, C + D)
    return (new_xyz, new_points)


# speedup vs baseline: 29.8732x; 1.0355x over previous
import functools

import jax
import jax.numpy as jnp
from jax import lax
from jax.experimental import pallas as pl
from jax.experimental.pallas import tpu as pltpu
from jax.experimental.pallas import tpu_sc as plsc


def _make_sc_concat(B, C, D, NH, NL):
    mesh = plsc.VectorSubcoreMesh(core_axis_name="c", subcore_axis_name="s")
    HH = NH // 2

    @functools.partial(
        pl.kernel,
        mesh=mesh,
        out_type=jax.ShapeDtypeStruct((B, C + D, NH, NL), jnp.float32),
        scratch_types=[
            pltpu.VMEM((3, HH, 8, NL), jnp.float32),
            pltpu.VMEM((C, NH, NL), jnp.float32),
            pltpu.SemaphoreType.DMA,
            pltpu.SemaphoreType.DMA,
            pltpu.SemaphoreType.DMA,
            pltpu.SemaphoreType.DMA,
            pltpu.SemaphoreType.DMA,
            pltpu.SemaphoreType.DMA,
            pltpu.SemaphoreType.DMA,
            pltpu.SemaphoreType.DMA,
        ],
        compiler_params=pltpu.CompilerParams(use_tc_tiling_on_sc=False),
    )
    def sc_concat(x_hbm, p_hbm, o_hbm, pbuf, xbuf, l0, l1, l2, s0, s1, s2, lx, sx):
        w = lax.axis_index("s") * 2 + lax.axis_index("c")
        ld = [l0, l1, l2]
        st = [s0, s1, s2]
        NB = 3
        n_chunks = 16

        def load(k):
            ci, h = k // 2, k % 2
            cp = pltpu.make_async_copy(
                p_hbm.at[w, ci, pl.ds(HH * h, HH), :, :],
                pbuf.at[k % NB],
                ld[k % NB],
            )
            cp.start()
            return cp

        xloads = []
        for c in range(C):
            cp = pltpu.make_async_copy(
                x_hbm.at[c, w // 8, :, w % 8, :], xbuf.at[c], lx
            )
            cp.start()
            xloads.append(cp)

        loads = {k: load(k) for k in range(NB - 1)}
        stores = {i: [] for i in range(NB)}
        for k in range(n_chunks):
            if k + NB - 1 < n_chunks:
                for q in stores[(k + NB - 1) % NB]:
                    q.wait()
                stores[(k + NB - 1) % NB] = []
                loads[k + NB - 1] = load(k + NB - 1)
            loads[k].wait()
            ci, h = k // 2, k % 2
            for cj in range(8):
                cp = pltpu.make_async_copy(
                    pbuf.at[k % NB, :, cj, :],
                    o_hbm.at[w, C + 8 * ci + cj, pl.ds(HH * h, HH), :],
                    st[k % NB],
                )
                cp.start()
                stores[k % NB].append(cp)
        xstores = []
        for c in range(C):
            xloads[c].wait()
            cp = pltpu.make_async_copy(xbuf.at[c], o_hbm.at[w, c, :, :], sx)
            cp.start()
            xstores.append(cp)
        for k in range(NB):
            for q in stores[k]:
                q.wait()
        for q in xstores:
            q.wait()

    return sc_concat


def kernel(xyz, points):
    B, N, C = xyz.shape
    D = points.shape[-1]
    NH, NL = N // 128, 128
    x_v = xyz.reshape(4, 8, NH, NL, C).transpose(4, 0, 2, 1, 3)
    p_v = points.reshape(B, NH, NL, 8, 8).transpose(0, 3, 1, 4, 2)

    out = _make_sc_concat(B, C, D, NH, NL)(x_v, p_v)

    new_xyz = jnp.zeros((B, 1, C), dtype=xyz.dtype)
    new_points = out.transpose(0, 2, 3, 1).reshape(B, 1, N, C + D)
    return (new_xyz, new_points)


# speedup vs baseline: 30.2110x; 1.0113x over previous
import functools

import jax
import jax.numpy as jnp
from jax import lax
from jax.experimental import pallas as pl
from jax.experimental.pallas import tpu as pltpu
from jax.experimental.pallas import tpu_sc as plsc


def _make_sc_concat(B, C, D, NH, NL):
    mesh = plsc.VectorSubcoreMesh(core_axis_name="c", subcore_axis_name="s")
    HH = NH // 2

    @functools.partial(
        pl.kernel,
        mesh=mesh,
        out_type=jax.ShapeDtypeStruct((B, C + D, NH, NL), jnp.float32),
        scratch_types=[
            pltpu.VMEM((3, 4, NH, NL), jnp.float32),
            pltpu.VMEM((C, NH, NL), jnp.float32),
            pltpu.SemaphoreType.DMA,
            pltpu.SemaphoreType.DMA,
            pltpu.SemaphoreType.DMA,
            pltpu.SemaphoreType.DMA,
            pltpu.SemaphoreType.DMA,
            pltpu.SemaphoreType.DMA,
            pltpu.SemaphoreType.DMA,
            pltpu.SemaphoreType.DMA,
        ],
        compiler_params=pltpu.CompilerParams(use_tc_tiling_on_sc=False),
    )
    def sc_concat(x_hbm, p_hbm, o_hbm, pbuf, xbuf, l0, l1, l2, s0, s1, s2, lx, sx):
        w = lax.axis_index("s") * 2 + lax.axis_index("c")
        ld = [l0, l1, l2]
        st = [s0, s1, s2]
        NB = 3
        n_chunks = 16

        def load(k):
            ci, cjh = k // 2, 4 * (k % 2)
            cps = []
            for j in range(4):
                cp = pltpu.make_async_copy(
                    p_hbm.at[w, ci, :, cjh + j, :],
                    pbuf.at[k % NB, j],
                    ld[k % NB],
                )
                cp.start()
                cps.append(cp)
            return cps

        xloads = []
        for c in range(C):
            cp = pltpu.make_async_copy(
                x_hbm.at[c, w // 8, :, w % 8, :], xbuf.at[c], lx
            )
            cp.start()
            xloads.append(cp)

        loads = {k: load(k) for k in range(NB - 1)}
        stores = {i: [] for i in range(NB)}
        for k in range(n_chunks):
            if k + NB - 1 < n_chunks:
                for q in stores[(k + NB - 1) % NB]:
                    q.wait()
                stores[(k + NB - 1) % NB] = []
                loads[k + NB - 1] = load(k + NB - 1)
            for q in loads[k]:
                q.wait()
            ci, cjh = k // 2, 4 * (k % 2)
            cp = pltpu.make_async_copy(
                pbuf.at[k % NB],
                o_hbm.at[w, pl.ds(C + 8 * ci + cjh, 4), :, :],
                st[k % NB],
            )
            cp.start()
            stores[k % NB].append(cp)
        xstores = []
        for c in range(C):
            xloads[c].wait()
            cp = pltpu.make_async_copy(xbuf.at[c], o_hbm.at[w, c, :, :], sx)
            cp.start()
            xstores.append(cp)
        for k in range(NB):
            for q in stores[k]:
                q.wait()
        for q in xstores:
            q.wait()

    return sc_concat


def kernel(xyz, points):
    B, N, C = xyz.shape
    D = points.shape[-1]
    NH, NL = N // 128, 128
    x_v = xyz.reshape(4, 8, NH, NL, C).transpose(4, 0, 2, 1, 3)
    p_v = points.reshape(B, NH, NL, 8, 8).transpose(0, 3, 1, 4, 2)

    out = _make_sc_concat(B, C, D, NH, NL)(x_v, p_v)

    new_xyz = jnp.zeros((B, 1, C), dtype=xyz.dtype)
    new_points = out.transpose(0, 2, 3, 1).reshape(B, 1, N, C + D)
    return (new_xyz, new_points)


# speedup vs baseline: 30.2181x; 1.0002x over previous
import functools

import jax
import jax.numpy as jnp
from jax import lax
from jax.experimental import pallas as pl
from jax.experimental.pallas import tpu as pltpu
from jax.experimental.pallas import tpu_sc as plsc


def _make_sc_concat(B, C, D, NH, NL):
    mesh = plsc.VectorSubcoreMesh(core_axis_name="c", subcore_axis_name="s")

    @functools.partial(
        pl.kernel,
        mesh=mesh,
        out_type=jax.ShapeDtypeStruct((B, C + D, NH, NL), jnp.float32),
        scratch_types=[
            pltpu.VMEM((3, 4, NH, NL), jnp.float32),
            pltpu.VMEM((C, NH, NL), jnp.float32),
            pltpu.SemaphoreType.DMA,
            pltpu.SemaphoreType.DMA,
            pltpu.SemaphoreType.DMA,
            pltpu.SemaphoreType.DMA,
            pltpu.SemaphoreType.DMA,
            pltpu.SemaphoreType.DMA,
            pltpu.SemaphoreType.DMA,
            pltpu.SemaphoreType.DMA,
        ],
        compiler_params=pltpu.CompilerParams(use_tc_tiling_on_sc=False),
    )
    def sc_concat(x_hbm, p_hbm, o_hbm, pbuf, xbuf, l0, l1, l2, s0, s1, s2, lx, sx):
        w = lax.axis_index("s") * 2 + lax.axis_index("c")
        ld = [l0, l1, l2]
        st = [s0, s1, s2]
        NB = 3
        n_chunks = D // 4

        def load(k):
            ci, cjh = k // 2, 4 * (k % 2)
            cps = []
            for j in range(4):
                cp = pltpu.make_async_copy(
                    p_hbm.at[w, ci, :, cjh + j, :],
                    pbuf.at[k % NB, j],
                    ld[k % NB],
                )
                cp.start()
                cps.append(cp)
            return cps

        xloads = []
        for c in range(C):
            cp = pltpu.make_async_copy(
                x_hbm.at[c, w // 8, :, w % 8, :], xbuf.at[c], lx
            )
            cp.start()
            xloads.append(cp)

        loads = {k: load(k) for k in range(NB - 1)}
        stores = {i: [] for i in range(NB)}
        for k in range(n_chunks):
            if k + NB - 1 < n_chunks:
                for q in stores[(k + NB - 1) % NB]:
                    q.wait()
                stores[(k + NB - 1) % NB] = []
                loads[k + NB - 1] = load(k + NB - 1)
            for q in loads[k]:
                q.wait()
            ci, cjh = k // 2, 4 * (k % 2)
            cp = pltpu.make_async_copy(
                pbuf.at[k % NB],
                o_hbm.at[w, pl.ds(C + 8 * ci + cjh, 4), :, :],
                st[k % NB],
            )
            cp.start()
            stores[k % NB].append(cp)
        xstores = []
        for c in range(C):
            xloads[c].wait()
            cp = pltpu.make_async_copy(xbuf.at[c], o_hbm.at[w, c, :, :], sx)
            cp.start()
            xstores.append(cp)
        for k in range(NB):
            for q in stores[k]:
                q.wait()
        for q in xstores:
            q.wait()

    return sc_concat


def kernel(xyz, points):
    B, N, C = xyz.shape
    D = points.shape[-1]
    NH, NL = N // 128, 128
    x_v = xyz.reshape(4, 8, NH, NL, C).transpose(4, 0, 2, 1, 3)
    p_v = points.reshape(B, NH, NL, 8, 8).transpose(0, 3, 1, 4, 2)

    out = _make_sc_concat(B, C, D, NH, NL)(x_v, p_v)

    new_xyz = jnp.zeros((B, 1, C), dtype=xyz.dtype)
    new_points = out.transpose(0, 2, 3, 1).reshape(B, 1, N, C + D)
    return (new_xyz, new_points)
